# baseline (speedup 1.0000x reference)
import functools

import jax
import jax.numpy as jnp
from jax import lax
from jax.experimental import pallas as pl
from jax.experimental.pallas import tpu as pltpu
from jax.experimental.pallas import tpu_sc as plsc

D = 64
B = 16384
J = 50
NC, NS = 2, 16
NW = NC * NS
CH = 128
IB_PER_W = (B // CH) // NW
NBUF = 4

_mesh = plsc.VectorSubcoreMesh(core_axis_name="c", subcore_axis_name="s")


@functools.partial(
    pl.kernel,
    mesh=_mesh,
    out_type=jax.ShapeDtypeStruct((J, D, B), jnp.float32),
    compiler_params=pltpu.CompilerParams(
        use_tc_tiling_on_sc=True, needs_layout_passes=False),
    scratch_types=[
        pltpu.VMEM((NBUF, CH), jnp.int32),
        pltpu.VMEM((NBUF, CH), jnp.int32),
        pltpu.VMEM((NBUF, CH, 128), jnp.float32),
        pltpu.VMEM((D, CH), jnp.float32),
    ] + [pltpu.SemaphoreType.DMA] * NBUF,
)
def _embed_kernel(table2_hbm, xt_hbm, out_hbm, raw_v, q_v, rows_v, tr_v,
                  *sems):
    wid = lax.axis_index("s") * NC + lax.axis_index("c")
    lane = jnp.arange(16, dtype=jnp.int32)

    def load_idx(b, j, bi):
        i0 = (wid * IB_PER_W + bi) * CH
        pltpu.sync_copy(xt_hbm.at[j, pl.ds(i0, CH)], raw_v.at[b])
        for k in range(CH // 16):
            v = raw_v[b, pl.ds(k * 16, 16)]
            q_v[b, pl.ds(k * 16, 16)] = lax.shift_right_logical(v, 1)

    def start(b):
        pltpu.make_async_copy(
            table2_hbm.at[q_v.at[b]], rows_v.at[b], sems[b]).start()

    def wait(b):
        pltpu.make_async_copy(
            table2_hbm.at[q_v.at[b]], rows_v.at[b], sems[b]).wait()

    for b in range(NBUF):
        load_idx(b, 0, b)
        start(b)

    def body(j, carry):
        for b in range(NBUF):
            wait(b)
            rowvecs = [lane + (g * 16) for g in range(CH // 16)]
            colbase = [
                (raw_v[b, pl.ds(g * 16, 16)] & 1) * D
                for g in range(CH // 16)
            ]

            def dstep(d, c):
                for g in range(CH // 16):
                    val = plsc.load_gather(
                        rows_v.at[b], [rowvecs[g], colbase[g] + d])
                    tr_v[d, pl.ds(g * 16, 16)] = val
                return c

            lax.fori_loop(0, D, dstep, 0)
            i0 = (wid * IB_PER_W + b) * CH
            pltpu.sync_copy(tr_v, out_hbm.at[j, :, pl.ds(i0, CH)])

            @pl.when(j < J - 1)
            def _():
                load_idx(b, j + 1, b)
                start(b)
        return carry

    lax.fori_loop(0, J, body, 0)


def kernel(x, table):
    table2 = table.reshape(500000, 128)
    xt = x.T.astype(jnp.int32)
    out_t = _embed_kernel(table2, xt)
    return out_t.transpose(2, 0, 1)
